# baseline (speedup 1.0000x reference)
import functools

import jax
import jax.numpy as jnp
from jax import lax
from jax.experimental import pallas as pl
from jax.experimental.pallas import tpu as pltpu
from jax.experimental.pallas import tpu_sc as plsc

K = 16
RB = 128
TC_CB = 256
SC_SCH = 128
SC_CHANNELS = 2048


def _bitonic_merge(vals, desc):
    n = len(vals)
    if n == 1:
        return vals
    half = n // 2
    out = list(vals)
    for i in range(half):
        hi = jnp.maximum(vals[i], vals[i + half])
        lo = jnp.minimum(vals[i], vals[i + half])
        if desc:
            out[i], out[i + half] = hi, lo
        else:
            out[i], out[i + half] = lo, hi
    return _bitonic_merge(out[:half], desc) + _bitonic_merge(out[half:], desc)


def _oe_merge(a, b, desc):
    if len(a) == 1 and len(b) == 1:
        hi = jnp.maximum(a[0], b[0])
        lo = jnp.minimum(a[0], b[0])
        return [hi, lo] if desc else [lo, hi]
    even = _oe_merge(a[0::2], b[0::2], desc)
    odd = _oe_merge(a[1::2], b[1::2], desc)
    out = [even[0]]
    for i in range(len(odd) - 1):
        hi = jnp.maximum(odd[i], even[i + 1])
        lo = jnp.minimum(odd[i], even[i + 1])
        if desc:
            out.extend([hi, lo])
        else:
            out.extend([lo, hi])
    out.append(odd[-1])
    return out


def _oe_sort(vals, desc):
    n = len(vals)
    if n == 1:
        return vals
    half = n // 2
    return _oe_merge(_oe_sort(vals[:half], desc), _oe_sort(vals[half:], desc), desc)


def _merge_topk(acc, vals):
    merged = [jnp.maximum(acc[j], vals[K - 1 - j]) for j in range(K)]
    return _bitonic_merge(merged, True)


def _tc_body(x_ref, o_ref):
    s = x_ref.shape[1]
    cb = x_ref.shape[2]
    n_chunks = s // RB

    def body(i, acc):
        chunk = x_ref[0, pl.ds(i * RB, RB), :]
        c3 = chunk.reshape(K, 8, cb)
        vals = _oe_sort([c3[j] for j in range(K)], True)
        return tuple(_merge_topk(acc, vals))

    acc0 = tuple(jnp.full((8, cb), -jnp.inf, jnp.float32) for _ in range(K))
    acc = lax.fori_loop(0, n_chunks, body, acc0, unroll=4)

    lists = list(acc)
    w = 8
    while w > 1:
        half = w // 2
        a = [v[:half] for v in lists]
        b = [v[half:] for v in lists]
        merged = [jnp.maximum(a[j], b[K - 1 - j]) for j in range(K)]
        lists = _bitonic_merge(merged, True)
        w = half
    o_ref[0] = jnp.concatenate(lists, axis=0)


def _tc_topk(inputs, c_lo, c_hi):
    b, s, c = inputs.shape
    cb = min(c_hi - c_lo, TC_CB)
    off = c_lo // cb
    grid = (b, (c_hi - c_lo) // cb)
    return pl.pallas_call(
        _tc_body,
        grid=grid,
        in_specs=[pl.BlockSpec((1, s, cb), lambda i, j: (i, 0, j + off))],
        out_specs=pl.BlockSpec((1, K, cb), lambda i, j: (i, 0, j)),
        out_shape=jax.ShapeDtypeStruct((b, K, c_hi - c_lo), jnp.float32),
    )(inputs)


def _make_sc_topk(B, S, C, c_base, c_span):
    NW = 32
    n_slices = NW // B
    cbs = c_span // n_slices
    assert cbs % 16 == 0
    n_chunks = S // SC_SCH
    mesh = plsc.VectorSubcoreMesh(core_axis_name="c", subcore_axis_name="s",
                                  num_cores=2, num_subcores=16)

    @functools.partial(
        pl.kernel,
        out_type=jax.ShapeDtypeStruct((B, K, c_span), jnp.float32),
        mesh=mesh,
        scratch_types=[
            pltpu.VMEM((2, SC_SCH, cbs), jnp.float32),
            pltpu.VMEM((K, cbs), jnp.float32),
            pltpu.SemaphoreType.DMA,
            pltpu.SemaphoreType.DMA,
        ],
    )
    def sc_topk(x_hbm, out_hbm, bufs, accb, sem0, sem1):
        wid = lax.axis_index("s") * 2 + lax.axis_index("c")
        b = wid // n_slices
        co = (wid % n_slices) * cbs
        c0 = c_base + co

        def src(chunk):
            return x_hbm.at[b, pl.ds(chunk * SC_SCH, SC_SCH), pl.ds(c0, cbs)]

        neg = jnp.full((16,), -jnp.inf, dtype=jnp.float32)

        def init_g(g, _):
            for j in range(K):
                accb[j, pl.ds(g * 16, 16)] = neg
            return 0

        lax.fori_loop(0, cbs // 16, init_g, 0)

        sems = (sem0, sem1)
        pltpu.async_copy(src(0), bufs.at[0], sem0)

        def compute(buf):
            def g_body(g, _):
                gs = g * 16
                acc = tuple(accb[j, pl.ds(gs, 16)] for j in range(K))

                def s_body(sub, acc):
                    base = sub * 16
                    vals = [buf[base + r, pl.ds(gs, 16)] for r in range(K)]
                    return tuple(_merge_topk(acc, _oe_sort(vals, True)))

                acc = lax.fori_loop(0, SC_SCH // 16, s_body, acc)
                for j in range(K):
                    accb[j, pl.ds(gs, 16)] = acc[j]
                return 0

            lax.fori_loop(0, cbs // 16, g_body, 0)

        def outer(i2, _):
            for parity in range(2):
                chunk = i2 * 2 + parity
                pltpu.make_async_copy(src(chunk), bufs.at[parity],
                                      sems[parity]).wait()

                @pl.when(chunk + 1 < n_chunks)
                def _start_next():
                    pltpu.async_copy(src(chunk + 1), bufs.at[1 - parity],
                                     sems[1 - parity])

                compute(bufs.at[parity])
            return 0

        lax.fori_loop(0, n_chunks // 2, outer, 0)
        pltpu.sync_copy(accb, out_hbm.at[b, :, pl.ds(co, cbs)])

    return sc_topk


def kernel(inputs):
    b, s, c = inputs.shape
    c_tc = c - SC_CHANNELS
    parts = []
    if c_tc > 0:
        parts.append(_tc_topk(inputs, 0, c_tc))
    if c_tc < c:
        parts.append(_make_sc_topk(b, s, c, c_tc, c - c_tc)(inputs))
    out = parts[0] if len(parts) == 1 else jnp.concatenate(parts, axis=2)
    return jnp.transpose(out, (0, 2, 1)).reshape(b, c * K)


# speedup vs baseline: 1.8582x; 1.8582x over previous
import functools

import jax
import jax.numpy as jnp
from jax import lax
from jax.experimental import pallas as pl
from jax.experimental.pallas import tpu as pltpu
from jax.experimental.pallas import tpu_sc as plsc

K = 16
RB = 128
TC_CB = 256
SC_SCH = 128
SC_CHANNELS = 1024


def _bitonic_merge(vals, desc):
    n = len(vals)
    if n == 1:
        return vals
    half = n // 2
    out = list(vals)
    for i in range(half):
        hi = jnp.maximum(vals[i], vals[i + half])
        lo = jnp.minimum(vals[i], vals[i + half])
        if desc:
            out[i], out[i + half] = hi, lo
        else:
            out[i], out[i + half] = lo, hi
    return _bitonic_merge(out[:half], desc) + _bitonic_merge(out[half:], desc)


def _oe_merge(a, b, desc):
    if len(a) == 1 and len(b) == 1:
        hi = jnp.maximum(a[0], b[0])
        lo = jnp.minimum(a[0], b[0])
        return [hi, lo] if desc else [lo, hi]
    even = _oe_merge(a[0::2], b[0::2], desc)
    odd = _oe_merge(a[1::2], b[1::2], desc)
    out = [even[0]]
    for i in range(len(odd) - 1):
        hi = jnp.maximum(odd[i], even[i + 1])
        lo = jnp.minimum(odd[i], even[i + 1])
        if desc:
            out.extend([hi, lo])
        else:
            out.extend([lo, hi])
    out.append(odd[-1])
    return out


def _oe_sort(vals, desc):
    n = len(vals)
    if n == 1:
        return vals
    half = n // 2
    return _oe_merge(_oe_sort(vals[:half], desc), _oe_sort(vals[half:], desc), desc)


def _merge_topk(acc, vals):
    merged = [jnp.maximum(acc[j], vals[K - 1 - j]) for j in range(K)]
    return _bitonic_merge(merged, True)


def _tc_body(x_ref, o_ref):
    s = x_ref.shape[1]
    cb = x_ref.shape[2]
    n_chunks = s // RB

    def body(i, acc):
        chunk = x_ref[0, pl.ds(i * RB, RB), :]
        c3 = chunk.reshape(K, 8, cb)
        vals = _oe_sort([c3[j] for j in range(K)], True)
        return tuple(_merge_topk(acc, vals))

    acc0 = tuple(jnp.full((8, cb), -jnp.inf, jnp.float32) for _ in range(K))
    acc = lax.fori_loop(0, n_chunks, body, acc0, unroll=4)

    lists = list(acc)
    w = 8
    while w > 1:
        half = w // 2
        a = [v[:half] for v in lists]
        b = [v[half:] for v in lists]
        merged = [jnp.maximum(a[j], b[K - 1 - j]) for j in range(K)]
        lists = _bitonic_merge(merged, True)
        w = half
    o_ref[0] = jnp.concatenate(lists, axis=0)


def _tc_topk(inputs, c_lo, c_hi):
    b, s, c = inputs.shape
    cb = min(c_hi - c_lo, TC_CB)
    off = c_lo // cb
    grid = (b, (c_hi - c_lo) // cb)
    return pl.pallas_call(
        _tc_body,
        grid=grid,
        in_specs=[pl.BlockSpec((1, s, cb), lambda i, j: (i, 0, j + off))],
        out_specs=pl.BlockSpec((1, K, cb), lambda i, j: (i, 0, j)),
        out_shape=jax.ShapeDtypeStruct((b, K, c_hi - c_lo), jnp.float32),
    )(inputs)


def _make_sc_topk(B, S, C, c_base, c_span):
    NW = 32
    n_slices = NW // B
    cbs = c_span // n_slices
    assert cbs % 16 == 0
    n_chunks = S // SC_SCH
    mesh = plsc.VectorSubcoreMesh(core_axis_name="c", subcore_axis_name="s",
                                  num_cores=2, num_subcores=16)

    @functools.partial(
        pl.kernel,
        out_type=jax.ShapeDtypeStruct((B, K, c_span), jnp.float32),
        mesh=mesh,
        scratch_types=[
            pltpu.VMEM((2, SC_SCH, cbs), jnp.float32),
            pltpu.VMEM((K, cbs), jnp.float32),
            pltpu.SemaphoreType.DMA,
            pltpu.SemaphoreType.DMA,
        ],
    )
    def sc_topk(x_hbm, out_hbm, bufs, accb, sem0, sem1):
        wid = lax.axis_index("s") * 2 + lax.axis_index("c")
        b = wid // n_slices
        co = (wid % n_slices) * cbs
        c0 = c_base + co

        def src(chunk):
            return x_hbm.at[b, pl.ds(chunk * SC_SCH, SC_SCH), pl.ds(c0, cbs)]

        neg = jnp.full((16,), -jnp.inf, dtype=jnp.float32)

        def init_g(g, _):
            for j in range(K):
                accb[j, pl.ds(g * 16, 16)] = neg
            return 0

        lax.fori_loop(0, cbs // 16, init_g, 0)

        sems = (sem0, sem1)
        pltpu.async_copy(src(0), bufs.at[0], sem0)

        def compute(buf):
            def g_body(g, _):
                gs = g * 16
                acc = tuple(accb[j, pl.ds(gs, 16)] for j in range(K))

                def s_body(sub, acc):
                    base = sub * 16
                    vals = [buf[base + r, pl.ds(gs, 16)] for r in range(K)]
                    return tuple(_merge_topk(acc, _oe_sort(vals, True)))

                acc = lax.fori_loop(0, SC_SCH // 16, s_body, acc)
                for j in range(K):
                    accb[j, pl.ds(gs, 16)] = acc[j]
                return 0

            lax.fori_loop(0, cbs // 16, g_body, 0)

        def outer(i2, _):
            for parity in range(2):
                chunk = i2 * 2 + parity
                pltpu.make_async_copy(src(chunk), bufs.at[parity],
                                      sems[parity]).wait()

                @pl.when(chunk + 1 < n_chunks)
                def _start_next():
                    pltpu.async_copy(src(chunk + 1), bufs.at[1 - parity],
                                     sems[1 - parity])

                compute(bufs.at[parity])
            return 0

        lax.fori_loop(0, n_chunks // 2, outer, 0)
        pltpu.sync_copy(accb, out_hbm.at[b, :, pl.ds(co, cbs)])

    return sc_topk


def kernel(inputs):
    b, s, c = inputs.shape
    c_tc = c - SC_CHANNELS
    parts = []
    if c_tc > 0:
        parts.append(_tc_topk(inputs, 0, c_tc))
    if c_tc < c:
        parts.append(_make_sc_topk(b, s, c, c_tc, c - c_tc)(inputs))
    out = parts[0] if len(parts) == 1 else jnp.concatenate(parts, axis=2)
    return jnp.transpose(out, (0, 2, 1)).reshape(b, c * K)


# speedup vs baseline: 2.5616x; 1.3785x over previous
import functools

import jax
import jax.numpy as jnp
from jax import lax
from jax.experimental import pallas as pl
from jax.experimental.pallas import tpu as pltpu
from jax.experimental.pallas import tpu_sc as plsc

K = 16
RB = 128
TC_CB = 256
SC_SCH = 128
SC_CHANNELS = 512


def _bitonic_merge(vals, desc):
    n = len(vals)
    if n == 1:
        return vals
    half = n // 2
    out = list(vals)
    for i in range(half):
        hi = jnp.maximum(vals[i], vals[i + half])
        lo = jnp.minimum(vals[i], vals[i + half])
        if desc:
            out[i], out[i + half] = hi, lo
        else:
            out[i], out[i + half] = lo, hi
    return _bitonic_merge(out[:half], desc) + _bitonic_merge(out[half:], desc)


def _oe_merge(a, b, desc):
    if len(a) == 1 and len(b) == 1:
        hi = jnp.maximum(a[0], b[0])
        lo = jnp.minimum(a[0], b[0])
        return [hi, lo] if desc else [lo, hi]
    even = _oe_merge(a[0::2], b[0::2], desc)
    odd = _oe_merge(a[1::2], b[1::2], desc)
    out = [even[0]]
    for i in range(len(odd) - 1):
        hi = jnp.maximum(odd[i], even[i + 1])
        lo = jnp.minimum(odd[i], even[i + 1])
        if desc:
            out.extend([hi, lo])
        else:
            out.extend([lo, hi])
    out.append(odd[-1])
    return out


def _oe_sort(vals, desc):
    n = len(vals)
    if n == 1:
        return vals
    half = n // 2
    return _oe_merge(_oe_sort(vals[:half], desc), _oe_sort(vals[half:], desc), desc)


def _merge_topk(acc, vals):
    merged = [jnp.maximum(acc[j], vals[K - 1 - j]) for j in range(K)]
    return _bitonic_merge(merged, True)


def _tc_body(x_ref, o_ref):
    s = x_ref.shape[1]
    cb = x_ref.shape[2]
    n_chunks = s // RB

    def body(i, acc):
        chunk = x_ref[0, pl.ds(i * RB, RB), :]
        c3 = chunk.reshape(K, 8, cb)
        vals = _oe_sort([c3[j] for j in range(K)], True)
        return tuple(_merge_topk(acc, vals))

    acc0 = tuple(jnp.full((8, cb), -jnp.inf, jnp.float32) for _ in range(K))
    acc = lax.fori_loop(0, n_chunks, body, acc0, unroll=4)

    lists = list(acc)
    w = 8
    while w > 1:
        half = w // 2
        a = [v[:half] for v in lists]
        b = [v[half:] for v in lists]
        merged = [jnp.maximum(a[j], b[K - 1 - j]) for j in range(K)]
        lists = _bitonic_merge(merged, True)
        w = half
    o_ref[0] = jnp.concatenate(lists, axis=0)


def _tc_topk(inputs, c_lo, c_hi):
    b, s, c = inputs.shape
    cb = min(c_hi - c_lo, TC_CB)
    off = c_lo // cb
    grid = (b, (c_hi - c_lo) // cb)
    return pl.pallas_call(
        _tc_body,
        grid=grid,
        in_specs=[pl.BlockSpec((1, s, cb), lambda i, j: (i, 0, j + off))],
        out_specs=pl.BlockSpec((1, K, cb), lambda i, j: (i, 0, j)),
        out_shape=jax.ShapeDtypeStruct((b, K, c_hi - c_lo), jnp.float32),
    )(inputs)


def _make_sc_topk(B, S, C, c_base, c_span):
    NW = 32
    n_slices = c_span // 128
    wps = NW // (B * n_slices)
    cw = 128 // wps
    assert cw % 16 == 0
    n_chunks = S // SC_SCH
    mesh = plsc.VectorSubcoreMesh(core_axis_name="c", subcore_axis_name="s",
                                  num_cores=2, num_subcores=16)

    @functools.partial(
        pl.kernel,
        out_type=jax.ShapeDtypeStruct((B, n_slices, wps, K, cw), jnp.float32),
        mesh=mesh,
        scratch_types=[
            pltpu.VMEM((2, SC_SCH, 128), jnp.float32),
            pltpu.VMEM((K, cw), jnp.float32),
            pltpu.SemaphoreType.DMA,
            pltpu.SemaphoreType.DMA,
        ],
    )
    def sc_topk(x_hbm, out_hbm, bufs, accb, sem0, sem1):
        wid = lax.axis_index("s") * 2 + lax.axis_index("c")
        sid = wid // wps
        half = wid % wps
        b = sid // n_slices
        sl = sid % n_slices
        c0 = c_base + sl * 128
        po = half * cw

        def src(chunk):
            return x_hbm.at[b, pl.ds(chunk * SC_SCH, SC_SCH), pl.ds(c0, 128)]

        neg = jnp.full((16,), -jnp.inf, dtype=jnp.float32)

        def init_g(g, _):
            for j in range(K):
                accb[j, pl.ds(g * 16, 16)] = neg
            return 0

        lax.fori_loop(0, cw // 16, init_g, 0)

        sems = (sem0, sem1)
        pltpu.async_copy(src(0), bufs.at[0], sem0)

        def compute(buf):
            def g_body(g, _):
                gs = g * 16
                acc = tuple(accb[j, pl.ds(gs, 16)] for j in range(K))

                def s_body(sub, acc):
                    base = sub * 16
                    vals = [buf[base + r, pl.ds(po + gs, 16)] for r in range(K)]
                    return tuple(_merge_topk(acc, _oe_sort(vals, True)))

                acc = lax.fori_loop(0, SC_SCH // 16, s_body, acc)
                for j in range(K):
                    accb[j, pl.ds(gs, 16)] = acc[j]
                return 0

            lax.fori_loop(0, cw // 16, g_body, 0)

        def outer(i2, _):
            for parity in range(2):
                chunk = i2 * 2 + parity
                pltpu.make_async_copy(src(chunk), bufs.at[parity],
                                      sems[parity]).wait()

                @pl.when(chunk + 1 < n_chunks)
                def _start_next():
                    pltpu.async_copy(src(chunk + 1), bufs.at[1 - parity],
                                     sems[1 - parity])

                compute(bufs.at[parity])
            return 0

        lax.fori_loop(0, n_chunks // 2, outer, 0)
        pltpu.sync_copy(accb, out_hbm.at[b, sl, half])

    return sc_topk


def kernel(inputs):
    b, s, c = inputs.shape
    c_tc = c - SC_CHANNELS
    parts = []
    if c_tc > 0:
        parts.append(_tc_topk(inputs, 0, c_tc))
    if c_tc < c:
        sc = _make_sc_topk(b, s, c, c_tc, c - c_tc)(inputs)
        sc = jnp.transpose(sc, (0, 3, 1, 2, 4)).reshape(b, K, c - c_tc)
        parts.append(sc)
    out = parts[0] if len(parts) == 1 else jnp.concatenate(parts, axis=2)
    return jnp.transpose(out, (0, 2, 1)).reshape(b, c * K)


# speedup vs baseline: 2.9257x; 1.1422x over previous
import functools

import jax
import jax.numpy as jnp
from jax import lax
from jax.experimental import pallas as pl
from jax.experimental.pallas import tpu as pltpu
from jax.experimental.pallas import tpu_sc as plsc

K = 16
RB = 128
TC_CB = 256
SC_SCH = 128
SC_CHANNELS = 512


def _bitonic_merge(vals, desc):
    n = len(vals)
    if n == 1:
        return vals
    half = n // 2
    out = list(vals)
    for i in range(half):
        hi = jnp.maximum(vals[i], vals[i + half])
        lo = jnp.minimum(vals[i], vals[i + half])
        if desc:
            out[i], out[i + half] = hi, lo
        else:
            out[i], out[i + half] = lo, hi
    return _bitonic_merge(out[:half], desc) + _bitonic_merge(out[half:], desc)


def _oe_merge(a, b, desc):
    if len(a) == 1 and len(b) == 1:
        hi = jnp.maximum(a[0], b[0])
        lo = jnp.minimum(a[0], b[0])
        return [hi, lo] if desc else [lo, hi]
    even = _oe_merge(a[0::2], b[0::2], desc)
    odd = _oe_merge(a[1::2], b[1::2], desc)
    out = [even[0]]
    for i in range(len(odd) - 1):
        hi = jnp.maximum(odd[i], even[i + 1])
        lo = jnp.minimum(odd[i], even[i + 1])
        if desc:
            out.extend([hi, lo])
        else:
            out.extend([lo, hi])
    out.append(odd[-1])
    return out


def _oe_sort(vals, desc):
    n = len(vals)
    if n == 1:
        return vals
    half = n // 2
    return _oe_merge(_oe_sort(vals[:half], desc), _oe_sort(vals[half:], desc), desc)


def _merge_topk(acc, vals):
    merged = [jnp.maximum(acc[j], vals[K - 1 - j]) for j in range(K)]
    return _bitonic_merge(merged, True)


def _tc_body(x_ref, o_ref):
    s = x_ref.shape[1]
    cb = x_ref.shape[2]
    n_chunks = s // RB

    def body(i, acc):
        chunk = x_ref[0, pl.ds(i * RB, RB), :]
        c3 = chunk.reshape(K, 8, cb)
        vals = _oe_sort([c3[j] for j in range(K)], True)
        return tuple(_merge_topk(acc, vals))

    acc0 = tuple(jnp.full((8, cb), -jnp.inf, jnp.float32) for _ in range(K))
    acc = lax.fori_loop(0, n_chunks, body, acc0, unroll=4)

    lists = list(acc)
    w = 8
    while w > 1:
        half = w // 2
        a = [v[:half] for v in lists]
        b = [v[half:] for v in lists]
        merged = [jnp.maximum(a[j], b[K - 1 - j]) for j in range(K)]
        lists = _bitonic_merge(merged, True)
        w = half
    o_ref[0] = jnp.concatenate(lists, axis=0)


def _tc_topk(inputs, c_lo, c_hi):
    b, s, c = inputs.shape
    cb = min(c_hi - c_lo, TC_CB)
    off = c_lo // cb
    grid = (b, (c_hi - c_lo) // cb)
    return pl.pallas_call(
        _tc_body,
        grid=grid,
        in_specs=[pl.BlockSpec((1, s, cb), lambda i, j: (i, 0, j + off))],
        out_specs=pl.BlockSpec((1, K, cb), lambda i, j: (i, 0, j)),
        out_shape=jax.ShapeDtypeStruct((b, K, c_hi - c_lo), jnp.float32),
    )(inputs)


def _make_sc_topk(B, S, C, c_base, c_span):
    NW = 32
    n_slices = c_span // 128
    sh = NW // (B * n_slices)
    assert sh == 2
    rows = S // sh
    n_chunks = rows // SC_SCH
    mesh = plsc.VectorSubcoreMesh(core_axis_name="c", subcore_axis_name="s",
                                  num_cores=2, num_subcores=16)

    @functools.partial(
        pl.kernel,
        out_type=jax.ShapeDtypeStruct((B, n_slices, sh, K, 128), jnp.float32),
        mesh=mesh,
        scratch_types=[
            pltpu.VMEM((2, SC_SCH, 128), jnp.float32),
            pltpu.VMEM((K, 128), jnp.float32),
            pltpu.SemaphoreType.DMA,
            pltpu.SemaphoreType.DMA,
        ],
    )
    def sc_topk(x_hbm, out_hbm, bufs, accb, sem0, sem1):
        wid = lax.axis_index("s") * 2 + lax.axis_index("c")
        sid = wid // sh
        half = wid % sh
        b = sid // n_slices
        sl = sid % n_slices
        c0 = c_base + sl * 128
        r0 = half * rows

        def src(chunk):
            return x_hbm.at[b, pl.ds(r0 + chunk * SC_SCH, SC_SCH),
                            pl.ds(c0, 128)]

        pltpu.async_copy(src(0), bufs.at[0], sem0)

        neg = jnp.full((16,), -jnp.inf, dtype=jnp.float32)

        def init_g(g, _):
            for j in range(K):
                accb[j, pl.ds(g * 16, 16)] = neg
            return 0

        lax.fori_loop(0, 128 // 16, init_g, 0)

        sems = (sem0, sem1)

        def compute(buf):
            def g_body(g, _):
                gs = g * 16
                acc = tuple(accb[j, pl.ds(gs, 16)] for j in range(K))

                def s_body(sub, acc):
                    base = sub * 16
                    vals = [buf[base + r, pl.ds(gs, 16)] for r in range(K)]
                    return tuple(_merge_topk(acc, _oe_sort(vals, True)))

                acc = lax.fori_loop(0, SC_SCH // 16, s_body, acc)
                for j in range(K):
                    accb[j, pl.ds(gs, 16)] = acc[j]
                return 0

            lax.fori_loop(0, 128 // 16, g_body, 0)

        def outer(i2, _):
            for parity in range(2):
                chunk = i2 * 2 + parity
                pltpu.make_async_copy(src(chunk), bufs.at[parity],
                                      sems[parity]).wait()

                @pl.when(chunk + 1 < n_chunks)
                def _start_next():
                    pltpu.async_copy(src(chunk + 1), bufs.at[1 - parity],
                                     sems[1 - parity])

                compute(bufs.at[parity])
            return 0

        lax.fori_loop(0, n_chunks // 2, outer, 0)
        pltpu.sync_copy(accb, out_hbm.at[b, sl, half])

    return sc_topk


def _sc_merge_body(p_ref, o_ref):
    n_slices = p_ref.shape[1]
    for sl in range(n_slices):
        a = [p_ref[0, sl, 0, i, :] for i in range(K)]
        bv = [p_ref[0, sl, 1, i, :] for i in range(K)]
        merged = [jnp.maximum(a[i], bv[K - 1 - i]) for i in range(K)]
        lists = _bitonic_merge(merged, True)
        o_ref[0, :, pl.ds(sl * 128, 128)] = jnp.stack(lists, axis=0)


def _sc_merge(partials):
    b, n_slices, sh, k, w = partials.shape
    return pl.pallas_call(
        _sc_merge_body,
        grid=(b,),
        in_specs=[pl.BlockSpec((1, n_slices, sh, k, w), lambda i: (i, 0, 0, 0, 0))],
        out_specs=pl.BlockSpec((1, k, n_slices * w), lambda i: (i, 0, 0)),
        out_shape=jax.ShapeDtypeStruct((b, k, n_slices * w), jnp.float32),
    )(partials)


def kernel(inputs):
    b, s, c = inputs.shape
    c_tc = c - SC_CHANNELS
    parts = []
    if c_tc > 0:
        parts.append(_tc_topk(inputs, 0, c_tc))
    if c_tc < c:
        partials = _make_sc_topk(b, s, c, c_tc, c - c_tc)(inputs)
        parts.append(_sc_merge(partials))
    out = parts[0] if len(parts) == 1 else jnp.concatenate(parts, axis=2)
    return jnp.transpose(out, (0, 2, 1)).reshape(b, c * K)
